# Initial kernel scaffold; baseline (speedup 1.0000x reference)
import jax
import jax.numpy as jnp
from jax.experimental import pallas as pl


def kernel(fm, corners, scale):
    raise NotImplementedError("write your pallas kernel here")


# speedup vs baseline: 44.3187x; 44.3187x over previous
import functools

import jax
import jax.numpy as jnp
from jax import lax
from jax.experimental import pallas as pl
from jax.experimental.pallas import tpu as pltpu
from jax.experimental.pallas import tpu_sc as plsc

_D = _H = _W = 24
_C = 128
_L = 16
_CC = _C // _L
_NW = 32


def _ext(pv, k):
    return pv[k]


def _build_sc_kernel(num_units, n_prop):
    mesh = plsc.VectorSubcoreMesh(core_axis_name="c", subcore_axis_name="s")
    units_per_w = num_units // _NW
    slab_words = _H * _W * _L

    @functools.partial(
        pl.kernel,
        mesh=mesh,
        out_type=jax.ShapeDtypeStruct((num_units, 8, _L), jnp.float32),
        scratch_types=[
            pltpu.VMEM((slab_words,), jnp.float32),
            pltpu.VMEM((_L,), jnp.int32),
            pltpu.VMEM((8, _L), jnp.float32),
        ],
    )
    def k(fm_hbm, par_hbm, out_hbm, slab, pv_ref, acc):
        wid = lax.axis_index("s") * 2 + lax.axis_index("c")
        neg = jnp.full((_L,), -jnp.inf, jnp.float32)

        def unit_body(t, carry):
            u = t * _NW + wid
            pn = lax.shift_right_logical(u, 3)
            cc = lax.bitwise_and(u, jnp.int32(7))
            b = lax.shift_right_logical(pn, 6)
            pltpu.sync_copy(par_hbm.at[pn], pv_ref)
            pv = pv_ref[:]
            fm_base = (b * _CC + cc) * _D

            for o in range(8):
                acc[o] = neg

            for bd in range(2):
                sd = _ext(pv, 2 * bd)
                ld = _ext(pv, 2 * bd + 1)

                def d_body(d, c, bd=bd):
                    pltpu.sync_copy(fm_hbm.at[fm_base + d], slab)
                    for bh in range(2):
                        sh = _ext(pv, 4 + 2 * bh)
                        lh = _ext(pv, 5 + 2 * bh)

                        def h_body(h, c2, bd=bd, bh=bh):
                            base_h = h * (_W * _L)
                            for bw in range(2):
                                sw = _ext(pv, 8 + 2 * bw)
                                lw = _ext(pv, 9 + 2 * bw)

                                def w_body(w, m):
                                    return jnp.maximum(
                                        m, slab[pl.ds(base_h + w * _L, _L)])

                                m = lax.fori_loop(sw, sw + lw, w_body, neg)
                                o = bd * 4 + bh * 2 + bw
                                acc[o] = jnp.maximum(acc[o], m)
                            return c2

                        lax.fori_loop(sh, sh + lh, h_body, c)
                    return c

                lax.fori_loop(sd, sd + ld, d_body, 0)

            pltpu.sync_copy(acc, out_hbm.at[u])
            return carry

        lax.fori_loop(0, units_per_w, unit_body, 0)

    return k


def kernel(fm, corners, scale):
    B, C, D, H, W = fm.shape
    N = corners.shape[1]

    c32 = corners.astype(jnp.int32)
    p1 = jnp.clip(c32[:, :, 0, :] // scale, 0, 21)
    p2r = c32[:, :, 1, :] // scale
    p2 = jnp.where(p2r - p1 >= 2, p2r, p1 + 2)
    n = p2 - p1
    s0, s1 = p1, p1 + n // 2
    l0, l1 = (n + 1) // 2, n - n // 2
    pr = jnp.stack(
        [s0[..., 0], l0[..., 0], s1[..., 0], l1[..., 0],
         s0[..., 1], l0[..., 1], s1[..., 1], l1[..., 1],
         s0[..., 2], l0[..., 2], s1[..., 2], l1[..., 2]], axis=-1)
    params = jnp.concatenate(
        [pr, jnp.zeros((B, N, 4), jnp.int32)], axis=-1).reshape(B * N, _L)

    fm2 = (fm.reshape(B, _CC, _L, D, H, W)
           .transpose(0, 1, 3, 4, 5, 2)
           .reshape(B * _CC * D, H * W * _L))

    num_units = B * N * _CC
    out = _build_sc_kernel(num_units, B * N)(fm2, params)
    out = (out.reshape(B, N, _CC, 8, _L)
           .transpose(0, 1, 2, 4, 3)
           .reshape(B, N, C, 2, 2, 2))
    return out


# speedup vs baseline: 53.9321x; 1.2169x over previous
import functools

import jax
import jax.numpy as jnp
from jax import lax
from jax.experimental import pallas as pl
from jax.experimental.pallas import tpu as pltpu
from jax.experimental.pallas import tpu_sc as plsc

_D = _H = _W = 24
_C = 128
_L = 16
_CC = _C // _L
_NW = 32
_ROW = _W * _L
_SLAB = _H * _ROW


def _build_sc_kernel(num_units):
    mesh = plsc.VectorSubcoreMesh(core_axis_name="c", subcore_axis_name="s")
    units_per_w = num_units // _NW

    @functools.partial(
        pl.kernel,
        mesh=mesh,
        out_type=jax.ShapeDtypeStruct((num_units * 8 * _L,), jnp.float32),
        scratch_types=[
            pltpu.VMEM((num_units // _CC * _L,), jnp.int32),
            pltpu.VMEM((2 * _SLAB,), jnp.float32),
            pltpu.VMEM((8, _L), jnp.float32),
            pltpu.VMEM((units_per_w * 8 * _L,), jnp.float32),
            pltpu.SemaphoreType.DMA,
            pltpu.SemaphoreType.DMA,
            pltpu.SemaphoreType.DMA,
        ],
    )
    def k(fm_hbm, par_hbm, out_hbm, par_v, slab, acc, outbuf, sem0, sem1,
          osem):
        wid = lax.axis_index("s") * 2 + lax.axis_index("c")
        pltpu.sync_copy(par_hbm, par_v)
        neg = jnp.full((_L,), -jnp.inf, jnp.float32)

        def unit_body(t, carry):
            u = t * _NW + wid
            pn = lax.shift_right_logical(u, 3)
            cc = lax.bitwise_and(u, jnp.int32(7))
            b = lax.shift_right_logical(pn, 6)
            pv = par_v[pl.ds(pn * _L, _L)]
            sd0, ld0, sd1, ld1 = pv[0], pv[1], pv[2], pv[3]
            sh0, lh0, sh1, lh1 = pv[4], pv[5], pv[6], pv[7]
            sw0, lw0, sw1, lw1 = pv[8], pv[9], pv[10], pv[11]
            d_lo = sd0
            d_hi = sd1 + ld1
            e0d = sd0 + ld0
            nh = sh1 + lh1 - sh0
            row0 = (b * _CC + cc) * _D

            for o in range(8):
                acc[o] = neg

            def emit_variant(nrows):
                words = nrows * _ROW
                h0 = jnp.minimum(sh0, _H - nrows)
                sems = (sem0, sem1)

                def issue(d, par):
                    pltpu.async_copy(
                        fm_hbm.at[row0 + d, pl.ds(h0 * _ROW, words)],
                        slab.at[pl.ds(par * _SLAB, words)],
                        sems[par])

                def wait(par):
                    pltpu.make_async_copy(
                        fm_hbm.at[0, pl.ds(0, words)],
                        slab.at[pl.ds(par * _SLAB, words)],
                        sems[par]).wait()

                issue(d_lo, 0)

                def d_body(d, c):
                    par = lax.bitwise_and(d - d_lo, jnp.int32(1))

                    @pl.when(par == 0)
                    def _():
                        wait(0)

                        @pl.when(d + 1 < d_hi)
                        def _():
                            issue(d + 1, 1)

                    @pl.when(par == 1)
                    def _():
                        wait(1)

                        @pl.when(d + 1 < d_hi)
                        def _():
                            issue(d + 1, 0)

                    base_p = par * _SLAB - h0 * _ROW
                    in_d = (d < e0d, d >= sd1)
                    for bd in range(2):
                        @pl.when(in_d[bd])
                        def _(bd=bd):
                            for bh in range(2):
                                sh = sh0 if bh == 0 else sh1
                                lh = lh0 if bh == 0 else lh1

                                def h_body(h, c2, bd=bd, bh=bh):
                                    hb = base_p + h * _ROW
                                    for bw in range(2):
                                        sw = sw0 if bw == 0 else sw1
                                        lw = lw0 if bw == 0 else lw1

                                        def w_body(w, m):
                                            return jnp.maximum(
                                                m,
                                                slab[pl.ds(hb + w * _L, _L)])

                                        m = lax.fori_loop(
                                            sw, sw + lw, w_body, neg)
                                        o = bd * 4 + bh * 2 + bw
                                        acc[o] = jnp.maximum(acc[o], m)
                                    return c2

                                lax.fori_loop(sh, sh + lh, h_body, 0)
                    return c

                lax.fori_loop(d_lo, d_hi, d_body, 0)

            @pl.when(nh <= 8)
            def _():
                emit_variant(8)

            @pl.when(jnp.logical_and(nh > 8, nh <= 16))
            def _():
                emit_variant(16)

            @pl.when(nh > 16)
            def _():
                emit_variant(24)

            for o in range(8):
                outbuf[pl.ds(t * 128 + o * _L, _L)] = acc[o]
            pltpu.async_copy(
                outbuf.at[pl.ds(t * 128, 128)],
                out_hbm.at[pl.ds(u * 128, 128)],
                osem)
            return carry

        lax.fori_loop(0, units_per_w, unit_body, 0)

        def drain(t, carry):
            pltpu.make_async_copy(
                outbuf.at[pl.ds(0, 128)],
                out_hbm.at[pl.ds(0, 128)],
                osem).wait()
            return carry

        lax.fori_loop(0, units_per_w, drain, 0)

    return k


def kernel(fm, corners, scale):
    B, C, D, H, W = fm.shape
    N = corners.shape[1]

    c32 = corners.astype(jnp.int32)
    p1 = jnp.clip(c32[:, :, 0, :] // scale, 0, 21)
    p2r = c32[:, :, 1, :] // scale
    p2 = jnp.where(p2r - p1 >= 2, p2r, p1 + 2)
    n = p2 - p1
    s0, s1 = p1, p1 + n // 2
    l0, l1 = (n + 1) // 2, n - n // 2
    pr = jnp.stack(
        [s0[..., 0], l0[..., 0], s1[..., 0], l1[..., 0],
         s0[..., 1], l0[..., 1], s1[..., 1], l1[..., 1],
         s0[..., 2], l0[..., 2], s1[..., 2], l1[..., 2]], axis=-1)
    params = jnp.concatenate(
        [pr, jnp.zeros((B, N, 4), jnp.int32)], axis=-1).reshape(B * N * _L)

    fm2 = (fm.reshape(B, _CC, _L, D, H, W)
           .transpose(0, 1, 3, 4, 5, 2)
           .reshape(B * _CC * D, H * W * _L))

    num_units = B * N * _CC
    out = _build_sc_kernel(num_units)(fm2, params)
    out = (out.reshape(B, N, _CC, 8, _L)
           .transpose(0, 1, 2, 4, 3)
           .reshape(B, N, C, 2, 2, 2))
    return out


# speedup vs baseline: 62.1325x; 1.1520x over previous
import functools

import jax
import jax.numpy as jnp
from jax import lax
from jax.experimental import pallas as pl
from jax.experimental.pallas import tpu as pltpu
from jax.experimental.pallas import tpu_sc as plsc

_D = _H = _W = 24
_C = 128
_L = 16
_CC = _C // _L
_NW = 32
_ROW = _W * _L
_SLAB = _H * _ROW


def _tp_body(x_ref, o_ref):
    o_ref[0] = x_ref[0].T


def _tc_channel_minor(fm):
    B, C, D, H, W = fm.shape
    v = D * H * W
    x = fm.reshape(B * _CC, _L, v)
    out = pl.pallas_call(
        _tp_body,
        grid=(B * _CC,),
        in_specs=[pl.BlockSpec((1, _L, v), lambda i: (i, 0, 0))],
        out_specs=pl.BlockSpec((1, v, _L), lambda i: (i, 0, 0)),
        out_shape=jax.ShapeDtypeStruct((B * _CC, v, _L), jnp.float32),
    )(x)
    return out.reshape(B * _CC * D, H * W * _L)


def _build_sc_kernel(num_units):
    mesh = plsc.VectorSubcoreMesh(core_axis_name="c", subcore_axis_name="s")
    units_per_w = num_units // _NW

    @functools.partial(
        pl.kernel,
        mesh=mesh,
        out_type=jax.ShapeDtypeStruct((num_units * 8 * _L,), jnp.float32),
        scratch_types=[
            pltpu.VMEM((num_units // _CC * _L,), jnp.int32),
            pltpu.VMEM((2 * _SLAB,), jnp.float32),
            pltpu.VMEM((8, _L), jnp.float32),
            pltpu.VMEM((units_per_w * 8 * _L,), jnp.float32),
            pltpu.SemaphoreType.DMA,
            pltpu.SemaphoreType.DMA,
            pltpu.SemaphoreType.DMA,
        ],
    )
    def k(fm_hbm, par_hbm, out_hbm, par_v, slab, acc, outbuf, sem0, sem1,
          osem):
        wid = lax.axis_index("s") * 2 + lax.axis_index("c")
        pltpu.sync_copy(par_hbm, par_v)
        neg = jnp.full((_L,), -jnp.inf, jnp.float32)

        def unit_body(t, carry):
            u = t * _NW + wid
            pn = lax.shift_right_logical(u, 3)
            cc = lax.bitwise_and(u, jnp.int32(7))
            b = lax.shift_right_logical(pn, 6)
            pv = par_v[pl.ds(pn * _L, _L)]
            sd0, ld0, sd1, ld1 = pv[0], pv[1], pv[2], pv[3]
            sh0, lh0, sh1, lh1 = pv[4], pv[5], pv[6], pv[7]
            sw0, lw0, sw1, lw1 = pv[8], pv[9], pv[10], pv[11]
            d_lo = sd0
            d_hi = sd1 + ld1
            e0d = sd0 + ld0
            nh = sh1 + lh1 - sh0
            row0 = (b * _CC + cc) * _D

            for o in range(8):
                acc[o] = neg

            def emit_variant(nrows):
                words = nrows * _ROW
                h0 = jnp.minimum(sh0, _H - nrows)
                sems = (sem0, sem1)

                def issue(d, par):
                    pltpu.async_copy(
                        fm_hbm.at[row0 + d, pl.ds(h0 * _ROW, words)],
                        slab.at[pl.ds(par * _SLAB, words)],
                        sems[par])

                def wait(par):
                    pltpu.make_async_copy(
                        fm_hbm.at[0, pl.ds(0, words)],
                        slab.at[pl.ds(par * _SLAB, words)],
                        sems[par]).wait()

                issue(d_lo, 0)

                def d_body(d, c):
                    par = lax.bitwise_and(d - d_lo, jnp.int32(1))

                    @pl.when(par == 0)
                    def _():
                        wait(0)

                        @pl.when(d + 1 < d_hi)
                        def _():
                            issue(d + 1, 1)

                    @pl.when(par == 1)
                    def _():
                        wait(1)

                        @pl.when(d + 1 < d_hi)
                        def _():
                            issue(d + 1, 0)

                    base_p = par * _SLAB - h0 * _ROW
                    in_d = (d < e0d, d >= sd1)
                    for bd in range(2):
                        @pl.when(in_d[bd])
                        def _(bd=bd):
                            for bh in range(2):
                                sh = sh0 if bh == 0 else sh1
                                lh = lh0 if bh == 0 else lh1

                                def h_body(h, c2, bd=bd, bh=bh):
                                    hb = base_p + h * _ROW
                                    for bw in range(2):
                                        sw = sw0 if bw == 0 else sw1
                                        lw = lw0 if bw == 0 else lw1

                                        def w_body(w, m):
                                            return jnp.maximum(
                                                m,
                                                slab[pl.ds(hb + w * _L, _L)])

                                        m = lax.fori_loop(
                                            sw, sw + lw, w_body, neg)
                                        o = bd * 4 + bh * 2 + bw
                                        acc[o] = jnp.maximum(acc[o], m)
                                    return c2

                                lax.fori_loop(sh, sh + lh, h_body, 0)
                    return c

                lax.fori_loop(d_lo, d_hi, d_body, 0)

            @pl.when(nh <= 8)
            def _():
                emit_variant(8)

            @pl.when(jnp.logical_and(nh > 8, nh <= 16))
            def _():
                emit_variant(16)

            @pl.when(nh > 16)
            def _():
                emit_variant(24)

            for o in range(8):
                outbuf[pl.ds(t * 128 + o * _L, _L)] = acc[o]
            pltpu.async_copy(
                outbuf.at[pl.ds(t * 128, 128)],
                out_hbm.at[pl.ds(u * 128, 128)],
                osem)
            return carry

        lax.fori_loop(0, units_per_w, unit_body, 0)

        def drain(t, carry):
            pltpu.make_async_copy(
                outbuf.at[pl.ds(0, 128)],
                out_hbm.at[pl.ds(0, 128)],
                osem).wait()
            return carry

        lax.fori_loop(0, units_per_w, drain, 0)

    return k


def kernel(fm, corners, scale):
    B, C, D, H, W = fm.shape
    N = corners.shape[1]

    c32 = corners.astype(jnp.int32)
    p1 = jnp.clip(c32[:, :, 0, :] // scale, 0, 21)
    p2r = c32[:, :, 1, :] // scale
    p2 = jnp.where(p2r - p1 >= 2, p2r, p1 + 2)
    n = p2 - p1
    s0, s1 = p1, p1 + n // 2
    l0, l1 = (n + 1) // 2, n - n // 2
    pr = jnp.stack(
        [s0[..., 0], l0[..., 0], s1[..., 0], l1[..., 0],
         s0[..., 1], l0[..., 1], s1[..., 1], l1[..., 1],
         s0[..., 2], l0[..., 2], s1[..., 2], l1[..., 2]], axis=-1)
    params = jnp.concatenate(
        [pr, jnp.zeros((B, N, 4), jnp.int32)], axis=-1).reshape(B * N * _L)

    fm2 = _tc_channel_minor(fm)
    num_units = B * N * _CC
    out = _build_sc_kernel(num_units)(fm2, params)
    out = (out.reshape(B, N, _CC, 8, _L)
           .transpose(0, 1, 2, 4, 3)
           .reshape(B, N, C, 2, 2, 2))
    return out


# speedup vs baseline: 62.2659x; 1.0021x over previous
import functools

import jax
import jax.numpy as jnp
from jax import lax
from jax.experimental import pallas as pl
from jax.experimental.pallas import tpu as pltpu
from jax.experimental.pallas import tpu_sc as plsc

_D = _H = _W = 24
_C = 128
_L = 16
_CC = _C // _L
_NW = 32
_ROW = _W * _L
_SLAB = _H * _ROW


def _tp_body(x_ref, o_ref):
    o_ref[0] = x_ref[0].T


def _tc_channel_minor(fm):
    B, C, D, H, W = fm.shape
    v = D * H * W
    x = fm.reshape(B * _CC, _L, v)
    out = pl.pallas_call(
        _tp_body,
        grid=(B * _CC,),
        in_specs=[pl.BlockSpec((1, _L, v), lambda i: (i, 0, 0))],
        out_specs=pl.BlockSpec((1, v, _L), lambda i: (i, 0, 0)),
        out_shape=jax.ShapeDtypeStruct((B * _CC, v, _L), jnp.float32),
    )(x)
    return out.reshape(B * _CC * D, H * W * _L)


def _ot_body(x_ref, o_ref):
    o_ref[...] = jnp.swapaxes(x_ref[...], 2, 3)


def _tc_oct_minor(flat, B, N):
    x = flat.reshape(B * N, _CC, 8, _L)
    out = pl.pallas_call(
        _ot_body,
        grid=(1,),
        in_specs=[pl.BlockSpec((B * N, _CC, 8, _L), lambda i: (0, 0, 0, 0))],
        out_specs=pl.BlockSpec((B * N, _CC, _L, 8), lambda i: (0, 0, 0, 0)),
        out_shape=jax.ShapeDtypeStruct((B * N, _CC, _L, 8), jnp.float32),
    )(x)
    return out


def _build_sc_kernel(num_units):
    mesh = plsc.VectorSubcoreMesh(core_axis_name="c", subcore_axis_name="s")
    units_per_w = num_units // _NW

    @functools.partial(
        pl.kernel,
        mesh=mesh,
        out_type=jax.ShapeDtypeStruct((num_units * 8 * _L,), jnp.float32),
        scratch_types=[
            pltpu.VMEM((num_units // _CC * _L,), jnp.int32),
            pltpu.VMEM((4 * _SLAB + 64,), jnp.float32),
            pltpu.VMEM((8, _L), jnp.float32),
            pltpu.VMEM((units_per_w * 8 * _L,), jnp.float32),
            pltpu.SemaphoreType.DMA,
            pltpu.SemaphoreType.DMA,
            pltpu.SemaphoreType.DMA,
            pltpu.SemaphoreType.DMA,
            pltpu.SemaphoreType.DMA,
        ],
    )
    def k(fm_hbm, par_hbm, out_hbm, par_v, slab, acc, outbuf,
          sem0, sem1, sem2, sem3, osem):
        wid = lax.axis_index("s") * 2 + lax.axis_index("c")
        pltpu.sync_copy(par_hbm, par_v)
        neg = jnp.full((_L,), -jnp.inf, jnp.float32)
        sems = (sem0, sem1, sem2, sem3)

        def unit_scalars(t):
            u = t * _NW + wid
            pn = lax.shift_right_logical(u, 3)
            cc = lax.bitwise_and(u, jnp.int32(7))
            b = lax.shift_right_logical(pn, 6)
            pv = par_v[pl.ds(pn * _L, _L)]
            row0 = (b * _CC + cc) * _D
            return u, pv, row0

        def issue_first(t, buf):
            _, pv, row0 = unit_scalars(t)
            sh0, lh1, sh1 = pv[4], pv[7], pv[6]
            nh = sh1 + lh1 - sh0
            d0 = pv[0]

            def go(nrows):
                h0 = jnp.minimum(sh0, _H - nrows)
                pltpu.async_copy(
                    fm_hbm.at[row0 + d0, pl.ds(h0 * _ROW, nrows * _ROW)],
                    slab.at[pl.ds(buf * _SLAB, nrows * _ROW)],
                    sems[buf])

            @pl.when(nh <= 8)
            def _():
                go(8)

            @pl.when(jnp.logical_and(nh > 8, nh <= 16))
            def _():
                go(16)

            @pl.when(nh > 16)
            def _():
                go(24)

        def wmax(hb, sw, lw):
            w_end = sw + lw
            nch = lax.shift_right_logical(lw + 3, 2)

            def chunk(j, ms):
                m0, m1, m2, m3 = ms
                w0 = sw + j * 4
                base = hb + w0 * _L
                x0 = slab[pl.ds(base, _L)]
                x1 = slab[pl.ds(base + _L, _L)]
                x2 = slab[pl.ds(base + 2 * _L, _L)]
                x3 = slab[pl.ds(base + 3 * _L, _L)]
                m0 = jnp.maximum(m0, x0)
                m1 = jnp.maximum(m1, jnp.where(w0 + 1 < w_end, x1, neg))
                m2 = jnp.maximum(m2, jnp.where(w0 + 2 < w_end, x2, neg))
                m3 = jnp.maximum(m3, jnp.where(w0 + 3 < w_end, x3, neg))
                return (m0, m1, m2, m3)

            m0, m1, m2, m3 = lax.fori_loop(0, nch, chunk, (neg, neg, neg, neg))
            return jnp.maximum(jnp.maximum(m0, m1), jnp.maximum(m2, m3))

        def unit_body(t, pair, last):
            beta = 2 * pair
            other = 2 * (1 - pair)
            u, pv, row0 = unit_scalars(t)
            sd0, ld0, sd1, ld1 = pv[0], pv[1], pv[2], pv[3]
            sh0, lh0, sh1, lh1 = pv[4], pv[5], pv[6], pv[7]
            sw0, lw0, sw1, lw1 = pv[8], pv[9], pv[10], pv[11]
            d_lo = sd0
            d_hi = sd1 + ld1
            e0d = sd0 + ld0
            nh = sh1 + lh1 - sh0

            if not last:
                issue_first(t + 1, other)

            for o in range(8):
                acc[o] = neg

            def emit_variant(nrows):
                words = nrows * _ROW
                h0 = jnp.minimum(sh0, _H - nrows)

                def issue(d, buf):
                    pltpu.async_copy(
                        fm_hbm.at[row0 + d, pl.ds(h0 * _ROW, words)],
                        slab.at[pl.ds(buf * _SLAB, words)],
                        sems[buf])

                def wait(buf):
                    pltpu.make_async_copy(
                        fm_hbm.at[0, pl.ds(0, words)],
                        slab.at[pl.ds(buf * _SLAB, words)],
                        sems[buf]).wait()

                def d_body(d, c):
                    par = lax.bitwise_and(d - d_lo, jnp.int32(1))

                    @pl.when(par == 0)
                    def _():
                        wait(beta)

                        @pl.when(d + 1 < d_hi)
                        def _():
                            issue(d + 1, beta + 1)

                    @pl.when(par == 1)
                    def _():
                        wait(beta + 1)

                        @pl.when(d + 1 < d_hi)
                        def _():
                            issue(d + 1, beta)

                    base_p = (beta + par) * _SLAB - h0 * _ROW
                    in_d = (d < e0d, d >= sd1)
                    for bd in range(2):
                        @pl.when(in_d[bd])
                        def _(bd=bd):
                            for bh in range(2):
                                sh = sh0 if bh == 0 else sh1
                                lh = lh0 if bh == 0 else lh1

                                def h_body(h, c2, bd=bd, bh=bh):
                                    hb = base_p + h * _ROW
                                    for bw in range(2):
                                        sw = sw0 if bw == 0 else sw1
                                        lw = lw0 if bw == 0 else lw1
                                        m = wmax(hb, sw, lw)
                                        o = bd * 4 + bh * 2 + bw
                                        acc[o] = jnp.maximum(acc[o], m)
                                    return c2

                                lax.fori_loop(sh, sh + lh, h_body, 0)
                    return c

                lax.fori_loop(d_lo, d_hi, d_body, 0)

            @pl.when(nh <= 8)
            def _():
                emit_variant(8)

            @pl.when(jnp.logical_and(nh > 8, nh <= 16))
            def _():
                emit_variant(16)

            @pl.when(nh > 16)
            def _():
                emit_variant(24)

            for o in range(8):
                outbuf[pl.ds(t * 128 + o * _L, _L)] = acc[o]
            pltpu.async_copy(
                outbuf.at[pl.ds(t * 128, 128)],
                out_hbm.at[pl.ds(u * 128, 128)],
                osem)

        issue_first(0, 0)

        def pair_body(t2, carry):
            unit_body(2 * t2, 0, False)
            unit_body(2 * t2 + 1, 1, False)
            return carry

        lax.fori_loop(0, units_per_w // 2 - 1, pair_body, 0)
        unit_body(units_per_w - 2, 0, False)
        unit_body(units_per_w - 1, 1, True)

        def drain(t, carry):
            pltpu.make_async_copy(
                outbuf.at[pl.ds(0, 128)],
                out_hbm.at[pl.ds(0, 128)],
                osem).wait()
            return carry

        lax.fori_loop(0, units_per_w, drain, 0)

    return k


def kernel(fm, corners, scale):
    B, C, D, H, W = fm.shape
    N = corners.shape[1]

    c32 = corners.astype(jnp.int32)
    p1 = jnp.clip(c32[:, :, 0, :] // scale, 0, 21)
    p2r = c32[:, :, 1, :] // scale
    p2 = jnp.where(p2r - p1 >= 2, p2r, p1 + 2)
    n = p2 - p1
    s0, s1 = p1, p1 + n // 2
    l0, l1 = (n + 1) // 2, n - n // 2
    pr = jnp.stack(
        [s0[..., 0], l0[..., 0], s1[..., 0], l1[..., 0],
         s0[..., 1], l0[..., 1], s1[..., 1], l1[..., 1],
         s0[..., 2], l0[..., 2], s1[..., 2], l1[..., 2]], axis=-1)
    params = jnp.concatenate(
        [pr, jnp.zeros((B, N, 4), jnp.int32)], axis=-1).reshape(B * N * _L)

    fm2 = _tc_channel_minor(fm)
    num_units = B * N * _CC
    out = _build_sc_kernel(num_units)(fm2, params)
    out = _tc_oct_minor(out, B, N).reshape(B, N, C, 2, 2, 2)
    return out


# speedup vs baseline: 65.9852x; 1.0597x over previous
import functools

import jax
import jax.numpy as jnp
from jax import lax
from jax.experimental import pallas as pl
from jax.experimental.pallas import tpu as pltpu
from jax.experimental.pallas import tpu_sc as plsc

_D = _H = _W = 24
_C = 128
_L = 16
_CC = _C // _L
_NW = 32
_ROW = _W * _L
_SLAB = _H * _ROW


def _tp_body(x_ref, o_ref):
    o_ref[0] = x_ref[0].T


def _tc_channel_minor(fm):
    B, C, D, H, W = fm.shape
    v = D * H * W
    x = fm.reshape(B * _CC, _L, v)
    out = pl.pallas_call(
        _tp_body,
        grid=(B * _CC,),
        in_specs=[pl.BlockSpec((1, _L, v), lambda i: (i, 0, 0))],
        out_specs=pl.BlockSpec((1, v, _L), lambda i: (i, 0, 0)),
        out_shape=jax.ShapeDtypeStruct((B * _CC, v, _L), jnp.float32),
    )(x)
    return out.reshape(B * _CC * D * H * W * _L)


def _ot_body(x_ref, o_ref):
    o_ref[...] = jnp.swapaxes(x_ref[...], 2, 3)


def _tc_oct_minor(flat, B, N):
    x = flat.reshape(B * N, _CC, 8, _L)
    out = pl.pallas_call(
        _ot_body,
        grid=(1,),
        in_specs=[pl.BlockSpec((B * N, _CC, 8, _L), lambda i: (0, 0, 0, 0))],
        out_specs=pl.BlockSpec((B * N, _CC, _L, 8), lambda i: (0, 0, 0, 0)),
        out_shape=jax.ShapeDtypeStruct((B * N, _CC, _L, 8), jnp.float32),
    )(x)
    return out


def _build_sc_kernel(num_units):
    mesh = plsc.VectorSubcoreMesh(core_axis_name="c", subcore_axis_name="s")
    units_per_w = num_units // _NW

    @functools.partial(
        pl.kernel,
        mesh=mesh,
        out_type=jax.ShapeDtypeStruct((num_units * 8 * _L,), jnp.float32),
        scratch_types=[
            pltpu.VMEM((num_units // _CC * _L,), jnp.int32),
            pltpu.VMEM((4 * _SLAB + 64,), jnp.float32),
            pltpu.VMEM((8, _L), jnp.float32),
            pltpu.VMEM((units_per_w * 8 * _L,), jnp.float32),
            pltpu.SemaphoreType.DMA,
            pltpu.SemaphoreType.DMA,
            pltpu.SemaphoreType.DMA,
            pltpu.SemaphoreType.DMA,
            pltpu.SemaphoreType.DMA,
        ],
    )
    def k(fm_hbm, par_hbm, out_hbm, par_v, slab, acc, outbuf,
          sem0, sem1, sem2, sem3, osem):
        wid = lax.axis_index("s") * 2 + lax.axis_index("c")
        pltpu.sync_copy(par_hbm, par_v)
        neg = jnp.full((_L,), -jnp.inf, jnp.float32)
        sems = (sem0, sem1, sem2, sem3)

        def unit_scalars(t):
            u = t * _NW + wid
            pn = lax.shift_right_logical(u, 3)
            cc = lax.bitwise_and(u, jnp.int32(7))
            b = lax.shift_right_logical(pn, 6)
            pv = par_v[pl.ds(pn * _L, _L)]
            row0 = (b * _CC + cc) * _D
            return u, pv, row0

        def issue_first(t, buf):
            _, pv, row0 = unit_scalars(t)
            sh0, lh1, sh1 = pv[4], pv[7], pv[6]
            nh = sh1 + lh1 - sh0
            d0 = pv[0]

            def go(nrows):
                h0 = jnp.minimum(sh0, _H - nrows)
                pltpu.async_copy(
                    fm_hbm.at[pl.ds((row0 + d0) * _SLAB + h0 * _ROW,
                                    nrows * _ROW)],
                    slab.at[pl.ds(buf * _SLAB, nrows * _ROW)],
                    sems[buf])

            @pl.when(nh <= 8)
            def _():
                go(8)

            @pl.when(jnp.logical_and(nh > 8, nh <= 16))
            def _():
                go(16)

            @pl.when(nh > 16)
            def _():
                go(24)

        def wmax(hb, sw, lw):
            w_end = sw + lw
            nch = lax.shift_right_logical(lw + 3, 2)

            def chunk(j, ms):
                m0, m1, m2, m3 = ms
                w0 = sw + j * 4
                base = hb + w0 * _L
                x0 = slab[pl.ds(base, _L)]
                x1 = slab[pl.ds(base + _L, _L)]
                x2 = slab[pl.ds(base + 2 * _L, _L)]
                x3 = slab[pl.ds(base + 3 * _L, _L)]
                m0 = jnp.maximum(m0, x0)
                m1 = jnp.maximum(m1, jnp.where(w0 + 1 < w_end, x1, neg))
                m2 = jnp.maximum(m2, jnp.where(w0 + 2 < w_end, x2, neg))
                m3 = jnp.maximum(m3, jnp.where(w0 + 3 < w_end, x3, neg))
                return (m0, m1, m2, m3)

            m0, m1, m2, m3 = lax.fori_loop(0, nch, chunk, (neg, neg, neg, neg))
            return jnp.maximum(jnp.maximum(m0, m1), jnp.maximum(m2, m3))

        def unit_body(t, pair, last):
            beta = 2 * pair
            other = 2 * (1 - pair)
            u, pv, row0 = unit_scalars(t)
            sd0, ld0, sd1, ld1 = pv[0], pv[1], pv[2], pv[3]
            sh0, lh0, sh1, lh1 = pv[4], pv[5], pv[6], pv[7]
            sw0, lw0, sw1, lw1 = pv[8], pv[9], pv[10], pv[11]
            d_lo = sd0
            d_hi = sd1 + ld1
            e0d = sd0 + ld0
            nh = sh1 + lh1 - sh0

            if not last:
                issue_first(t + 1, other)

            for o in range(8):
                acc[o] = neg

            def emit_variant(nrows):
                words = nrows * _ROW
                h0 = jnp.minimum(sh0, _H - nrows)

                def issue(d, buf):
                    pltpu.async_copy(
                        fm_hbm.at[pl.ds((row0 + d) * _SLAB + h0 * _ROW,
                                        words)],
                        slab.at[pl.ds(buf * _SLAB, words)],
                        sems[buf])

                def wait(buf):
                    pltpu.make_async_copy(
                        fm_hbm.at[pl.ds(0, words)],
                        slab.at[pl.ds(buf * _SLAB, words)],
                        sems[buf]).wait()

                def d_body(d, c):
                    par = lax.bitwise_and(d - d_lo, jnp.int32(1))

                    @pl.when(par == 0)
                    def _():
                        wait(beta)

                        @pl.when(d + 1 < d_hi)
                        def _():
                            issue(d + 1, beta + 1)

                    @pl.when(par == 1)
                    def _():
                        wait(beta + 1)

                        @pl.when(d + 1 < d_hi)
                        def _():
                            issue(d + 1, beta)

                    base_p = (beta + par) * _SLAB - h0 * _ROW
                    in_d = (d < e0d, d >= sd1)
                    for bd in range(2):
                        @pl.when(in_d[bd])
                        def _(bd=bd):
                            for bh in range(2):
                                sh = sh0 if bh == 0 else sh1
                                lh = lh0 if bh == 0 else lh1

                                def h_body(h, c2, bd=bd, bh=bh):
                                    hb = base_p + h * _ROW
                                    for bw in range(2):
                                        sw = sw0 if bw == 0 else sw1
                                        lw = lw0 if bw == 0 else lw1
                                        m = wmax(hb, sw, lw)
                                        o = bd * 4 + bh * 2 + bw
                                        acc[o] = jnp.maximum(acc[o], m)
                                    return c2

                                lax.fori_loop(sh, sh + lh, h_body, 0)
                    return c

                lax.fori_loop(d_lo, d_hi, d_body, 0)

            @pl.when(nh <= 8)
            def _():
                emit_variant(8)

            @pl.when(jnp.logical_and(nh > 8, nh <= 16))
            def _():
                emit_variant(16)

            @pl.when(nh > 16)
            def _():
                emit_variant(24)

            for o in range(8):
                outbuf[pl.ds(t * 128 + o * _L, _L)] = acc[o]
            pltpu.async_copy(
                outbuf.at[pl.ds(t * 128, 128)],
                out_hbm.at[pl.ds(u * 128, 128)],
                osem)

        issue_first(0, 0)

        def pair_body(t2, carry):
            unit_body(2 * t2, 0, False)
            unit_body(2 * t2 + 1, 1, False)
            return carry

        lax.fori_loop(0, units_per_w // 2 - 1, pair_body, 0)
        unit_body(units_per_w - 2, 0, False)
        unit_body(units_per_w - 1, 1, True)

        def drain(t, carry):
            pltpu.make_async_copy(
                outbuf.at[pl.ds(0, 128)],
                out_hbm.at[pl.ds(0, 128)],
                osem).wait()
            return carry

        lax.fori_loop(0, units_per_w, drain, 0)

    return k


def kernel(fm, corners, scale):
    B, C, D, H, W = fm.shape
    N = corners.shape[1]

    c32 = corners.astype(jnp.int32)
    p1 = jnp.clip(c32[:, :, 0, :] // scale, 0, 21)
    p2r = c32[:, :, 1, :] // scale
    p2 = jnp.where(p2r - p1 >= 2, p2r, p1 + 2)
    n = p2 - p1
    s0, s1 = p1, p1 + n // 2
    l0, l1 = (n + 1) // 2, n - n // 2
    pr = jnp.stack(
        [s0[..., 0], l0[..., 0], s1[..., 0], l1[..., 0],
         s0[..., 1], l0[..., 1], s1[..., 1], l1[..., 1],
         s0[..., 2], l0[..., 2], s1[..., 2], l1[..., 2]], axis=-1)
    params = jnp.concatenate(
        [pr, jnp.zeros((B, N, 4), jnp.int32)], axis=-1).reshape(B * N * _L)

    fm2 = _tc_channel_minor(fm)
    num_units = B * N * _CC
    out = _build_sc_kernel(num_units)(fm2, params)
    out = _tc_oct_minor(out, B, N).reshape(B, N, C, 2, 2, 2)
    return out


# speedup vs baseline: 72.4272x; 1.0976x over previous
import functools

import jax
import jax.numpy as jnp
from jax import lax
from jax.experimental import pallas as pl
from jax.experimental.pallas import tpu as pltpu
from jax.experimental.pallas import tpu_sc as plsc

_D = _H = _W = 24
_C = 128
_L = 16
_CC = _C // _L
_NW = 32
_ROW = _W * _L
_SLAB = _H * _ROW


def _tp_body(x_ref, o_ref):
    v = _D * _H * _W
    o_ref[0] = x_ref[0, 0].reshape(_L, v).T


def _tc_channel_minor(fm):
    B, C, D, H, W = fm.shape
    v = D * H * W
    out = pl.pallas_call(
        _tp_body,
        grid=(B, _CC),
        in_specs=[pl.BlockSpec((1, 1, _L, D, H, W),
                               lambda b, cc: (b, cc, 0, 0, 0, 0))],
        out_specs=pl.BlockSpec((1, v, _L), lambda b, cc: (b * _CC + cc, 0, 0)),
        out_shape=jax.ShapeDtypeStruct((B * _CC, v, _L), jnp.float32),
    )(fm.reshape(B, C // _L, _L, D, H, W))
    return out.reshape(B * _CC * v * _L)


def _ot_body(x_ref, o_ref):
    o_ref[...] = jnp.swapaxes(x_ref[...], 2, 3)


def _tc_oct_minor(flat, B, N):
    x = flat.reshape(B * N, _CC, 8, _L)
    out = pl.pallas_call(
        _ot_body,
        grid=(1,),
        in_specs=[pl.BlockSpec((B * N, _CC, 8, _L), lambda i: (0, 0, 0, 0))],
        out_specs=pl.BlockSpec((B * N, _CC, _L, 8), lambda i: (0, 0, 0, 0)),
        out_shape=jax.ShapeDtypeStruct((B * N, _CC, _L, 8), jnp.float32),
    )(x)
    return out


def _build_sc_kernel(num_units):
    mesh = plsc.VectorSubcoreMesh(core_axis_name="c", subcore_axis_name="s")
    units_per_w = num_units // _NW

    @functools.partial(
        pl.kernel,
        mesh=mesh,
        out_type=jax.ShapeDtypeStruct((num_units * 8 * _L,), jnp.float32),
        scratch_types=[
            pltpu.VMEM((num_units // _CC * _L,), jnp.int32),
            pltpu.VMEM((4 * _SLAB + 64,), jnp.float32),
            pltpu.VMEM((8, _L), jnp.float32),
            pltpu.VMEM((units_per_w * 8 * _L,), jnp.float32),
            pltpu.SemaphoreType.DMA,
            pltpu.SemaphoreType.DMA,
            pltpu.SemaphoreType.DMA,
            pltpu.SemaphoreType.DMA,
            pltpu.SemaphoreType.DMA,
        ],
    )
    def k(fm_hbm, par_hbm, out_hbm, par_v, slab, acc, outbuf,
          sem0, sem1, sem2, sem3, osem):
        wid = lax.axis_index("s") * 2 + lax.axis_index("c")
        pltpu.sync_copy(par_hbm, par_v)
        neg = jnp.full((_L,), -jnp.inf, jnp.float32)
        sems = (sem0, sem1, sem2, sem3)

        def unit_scalars(t):
            u = t * _NW + wid
            pn = lax.shift_right_logical(u, 3)
            cc = lax.bitwise_and(u, jnp.int32(7))
            b = lax.shift_right_logical(pn, 6)
            pv = par_v[pl.ds(pn * _L, _L)]
            g = b * _CC + cc
            return u, pv, g

        def issue_first(t, buf):
            _, pv, g = unit_scalars(t)
            sh0, lh1, sh1 = pv[4], pv[7], pv[6]
            nh = sh1 + lh1 - sh0
            d0 = pv[0]

            def go(nrows):
                h0 = jnp.minimum(sh0, _H - nrows)
                pltpu.async_copy(
                    fm_hbm.at[pl.ds((g * _D + d0) * _SLAB + h0 * _ROW,
                                    nrows * _ROW)],
                    slab.at[pl.ds(buf * _SLAB, nrows * _ROW)],
                    sems[buf])

            @pl.when(nh <= 8)
            def _():
                go(8)

            @pl.when(jnp.logical_and(nh > 8, nh <= 16))
            def _():
                go(16)

            @pl.when(nh > 16)
            def _():
                go(24)

        def wmax(hb, sw, lw):
            w_end = sw + lw
            nch = lax.shift_right_logical(lw + 3, 2)

            def chunk(j, ms):
                m0, m1, m2, m3 = ms
                w0 = sw + j * 4
                base = hb + w0 * _L
                x0 = slab[pl.ds(base, _L)]
                x1 = slab[pl.ds(base + _L, _L)]
                x2 = slab[pl.ds(base + 2 * _L, _L)]
                x3 = slab[pl.ds(base + 3 * _L, _L)]
                m0 = jnp.maximum(m0, x0)
                m1 = jnp.maximum(m1, jnp.where(w0 + 1 < w_end, x1, neg))
                m2 = jnp.maximum(m2, jnp.where(w0 + 2 < w_end, x2, neg))
                m3 = jnp.maximum(m3, jnp.where(w0 + 3 < w_end, x3, neg))
                return (m0, m1, m2, m3)

            m0, m1, m2, m3 = lax.fori_loop(0, nch, chunk, (neg, neg, neg, neg))
            return jnp.maximum(jnp.maximum(m0, m1), jnp.maximum(m2, m3))

        def unit_body(t, pair, last):
            beta = 2 * pair
            other = 2 * (1 - pair)
            u, pv, g = unit_scalars(t)
            sd0, ld0, sd1, ld1 = pv[0], pv[1], pv[2], pv[3]
            sh0, lh0, sh1, lh1 = pv[4], pv[5], pv[6], pv[7]
            sw0, lw0, sw1, lw1 = pv[8], pv[9], pv[10], pv[11]
            d_lo = sd0
            d_hi = sd1 + ld1
            e0d = sd0 + ld0
            nh = sh1 + lh1 - sh0

            if not last:
                issue_first(t + 1, other)

            for o in range(8):
                acc[o] = neg

            def emit_variant(nrows):
                words = nrows * _ROW
                h0 = jnp.minimum(sh0, _H - nrows)

                def issue(d, buf):
                    pltpu.async_copy(
                        fm_hbm.at[pl.ds((g * _D + d) * _SLAB + h0 * _ROW,
                                        words)],
                        slab.at[pl.ds(buf * _SLAB, words)],
                        sems[buf])

                def wait(buf):
                    pltpu.make_async_copy(
                        fm_hbm.at[pl.ds(0, words)],
                        slab.at[pl.ds(buf * _SLAB, words)],
                        sems[buf]).wait()

                def d_body(d, c):
                    par = lax.bitwise_and(d - d_lo, jnp.int32(1))

                    @pl.when(par == 0)
                    def _():
                        wait(beta)

                        @pl.when(d + 1 < d_hi)
                        def _():
                            issue(d + 1, beta + 1)

                    @pl.when(par == 1)
                    def _():
                        wait(beta + 1)

                        @pl.when(d + 1 < d_hi)
                        def _():
                            issue(d + 1, beta)

                    base_p = (beta + par) * _SLAB - h0 * _ROW
                    in_d = (d < e0d, d >= sd1)
                    for bd in range(2):
                        @pl.when(in_d[bd])
                        def _(bd=bd):
                            for bh in range(2):
                                sh = sh0 if bh == 0 else sh1
                                lh = lh0 if bh == 0 else lh1

                                def h_body(h, c2, bd=bd, bh=bh):
                                    hb = base_p + h * _ROW
                                    for bw in range(2):
                                        sw = sw0 if bw == 0 else sw1
                                        lw = lw0 if bw == 0 else lw1
                                        m = wmax(hb, sw, lw)
                                        o = bd * 4 + bh * 2 + bw
                                        acc[o] = jnp.maximum(acc[o], m)
                                    return c2

                                lax.fori_loop(sh, sh + lh, h_body, 0)
                    return c

                lax.fori_loop(d_lo, d_hi, d_body, 0)

            @pl.when(nh <= 8)
            def _():
                emit_variant(8)

            @pl.when(jnp.logical_and(nh > 8, nh <= 16))
            def _():
                emit_variant(16)

            @pl.when(nh > 16)
            def _():
                emit_variant(24)

            for o in range(8):
                outbuf[pl.ds(t * 128 + o * _L, _L)] = acc[o]
            pltpu.async_copy(
                outbuf.at[pl.ds(t * 128, 128)],
                out_hbm.at[pl.ds(u * 128, 128)],
                osem)

        issue_first(0, 0)

        def pair_body(t2, carry):
            unit_body(2 * t2, 0, False)
            unit_body(2 * t2 + 1, 1, False)
            return carry

        lax.fori_loop(0, units_per_w // 2 - 1, pair_body, 0)
        unit_body(units_per_w - 2, 0, False)
        unit_body(units_per_w - 1, 1, True)

        def drain(t, carry):
            pltpu.make_async_copy(
                outbuf.at[pl.ds(0, 128)],
                out_hbm.at[pl.ds(0, 128)],
                osem).wait()
            return carry

        lax.fori_loop(0, units_per_w, drain, 0)

    return k


def kernel(fm, corners, scale):
    B, C, D, H, W = fm.shape
    N = corners.shape[1]

    c32 = corners.astype(jnp.int32)
    p1 = jnp.clip(c32[:, :, 0, :] // scale, 0, 21)
    p2r = c32[:, :, 1, :] // scale
    p2 = jnp.where(p2r - p1 >= 2, p2r, p1 + 2)
    n = p2 - p1
    s0, s1 = p1, p1 + n // 2
    l0, l1 = (n + 1) // 2, n - n // 2
    pr = jnp.stack(
        [s0[..., 0], l0[..., 0], s1[..., 0], l1[..., 0],
         s0[..., 1], l0[..., 1], s1[..., 1], l1[..., 1],
         s0[..., 2], l0[..., 2], s1[..., 2], l1[..., 2]], axis=-1)
    params = jnp.concatenate(
        [pr, jnp.zeros((B, N, 4), jnp.int32)], axis=-1).reshape(B * N * _L)

    fm2 = _tc_channel_minor(fm)
    num_units = B * N * _CC
    out = _build_sc_kernel(num_units)(fm2, params)
    out = _tc_oct_minor(out, B, N).reshape(B, N, C, 2, 2, 2)
    return out


# speedup vs baseline: 116.3891x; 1.6070x over previous
import functools

import jax
import jax.numpy as jnp
from jax import lax
from jax.experimental import pallas as pl
from jax.experimental.pallas import tpu as pltpu
from jax.experimental.pallas import tpu_sc as plsc

_D = _H = _W = 24
_C = 128
_L = 16
_CC = _C // _L
_NW = 32
_ROW = _W * _C
_SLAB = _H * _ROW
_JROWS = 8
_JWORDS = _JROWS * _ROW
_NRING = 4


def _tp_body(x_ref, o_ref):
    o_ref[0] = x_ref[0, :, 0].reshape(_C, _H * _W).T


def _tc_channel_minor(fm):
    B, C, D, H, W = fm.shape
    out = pl.pallas_call(
        _tp_body,
        grid=(B, D),
        in_specs=[pl.BlockSpec((1, C, 1, H, W),
                               lambda b, d: (b, 0, d, 0, 0))],
        out_specs=pl.BlockSpec((1, H * W, C), lambda b, d: (b * D + d, 0, 0)),
        out_shape=jax.ShapeDtypeStruct((B * D, H * W, C), jnp.float32),
    )(fm)
    return out.reshape(B * D * H * W * C)


def _ot_body(x_ref, o_ref):
    o_ref[...] = jnp.swapaxes(x_ref[...], 2, 3)


def _tc_oct_minor(flat, B, N):
    x = flat.reshape(B * N, _CC, 8, _L)
    out = pl.pallas_call(
        _ot_body,
        grid=(1,),
        in_specs=[pl.BlockSpec((B * N, _CC, 8, _L), lambda i: (0, 0, 0, 0))],
        out_specs=pl.BlockSpec((B * N, _CC, _L, 8), lambda i: (0, 0, 0, 0)),
        out_shape=jax.ShapeDtypeStruct((B * N, _CC, _L, 8), jnp.float32),
    )(x)
    return out


def _build_sc_kernel(num_props):
    mesh = plsc.VectorSubcoreMesh(core_axis_name="c", subcore_axis_name="s")
    units_per_w = num_props // _NW

    @functools.partial(
        pl.kernel,
        mesh=mesh,
        out_type=jax.ShapeDtypeStruct((num_props * 8 * _C,), jnp.float32),
        scratch_types=[
            pltpu.VMEM((num_props * _L,), jnp.int32),
            pltpu.VMEM((_NRING * _JWORDS + 64,), jnp.float32),
            pltpu.VMEM((8 * _C,), jnp.float32),
            pltpu.VMEM((units_per_w * 8 * _C,), jnp.float32),
        ] + [pltpu.SemaphoreType.DMA] * (_NRING + 1),
    )
    def k(fm_hbm, par_hbm, out_hbm, par_v, ring, acc, outbuf, *sems_all):
        sems = sems_all[:_NRING]
        osem = sems_all[_NRING]
        wid = lax.axis_index("s") * 2 + lax.axis_index("c")
        pltpu.sync_copy(par_hbm, par_v)
        neg = jnp.full((_L,), -jnp.inf, jnp.float32)
        i32 = jnp.int32

        def load_pv(t):
            ts = jnp.minimum(t, units_per_w - 1)
            pn = ts * _NW + wid
            b = lax.shift_right_logical(pn, 6)
            return par_v[pl.ds(pn * _L, _L)], b

        def nk_of(pv):
            nh = pv[6] + pv[7] - pv[4]
            return lax.shift_right_logical(nh + 7, 3)

        def h0_of(pv, kk):
            return jnp.minimum(pv[4] + kk * _JROWS, _H - _JROWS)

        def issue_job_static(b, d, h0, s):
            src = (b * _D + d) * _SLAB + h0 * _ROW
            pltpu.async_copy(
                fm_hbm.at[pl.ds(src, _JWORDS)],
                ring.at[pl.ds(s * _JWORDS, _JWORDS)],
                sems[s])

        def wait_job_static(s):
            pltpu.make_async_copy(
                fm_hbm.at[pl.ds(0, _JWORDS)],
                ring.at[pl.ds(s * _JWORDS, _JWORDS)],
                sems[s]).wait()

        def adv(t, d, kk, pv, b):
            k2 = kk + 1
            roll_k = k2 >= nk_of(pv)
            k3 = jnp.where(roll_k, 0, k2)
            d2 = jnp.where(roll_k, d + 1, d)
            d_hi = pv[2] + pv[3]
            roll_u = jnp.logical_and(roll_k, d2 >= d_hi)
            t2 = jnp.where(roll_u, t + 1, t)
            pv2, b2 = load_pv(t2)
            pv3 = jnp.where(roll_u, pv2, pv)
            b3 = jnp.where(roll_u, b2, b)
            d3 = jnp.where(roll_u, pv3[0], d2)
            return t2, d3, k3, pv3, b3, roll_u

        def wmax8(hb, sw, lw):
            w_end = sw + lw
            nch = lax.shift_right_logical(lw + 3, 2)

            def chunk(j, ms):
                w0 = sw + j * 4
                base = hb + w0 * _C
                out = []
                for cc in range(_CC):
                    m = ms[cc]
                    for e in range(4):
                        x = ring[pl.ds(base + e * _C + cc * _L, _L)]
                        if e == 0:
                            m = jnp.maximum(m, x)
                        else:
                            m = jnp.maximum(
                                m, jnp.where(w0 + e < w_end, x, neg))
                    out.append(m)
                return tuple(out)

            ms = lax.fori_loop(0, nch, chunk, (neg,) * _CC)
            return ms

        def compute_job(pv, d, kk, slot_idx):
            h0 = h0_of(pv, kk)
            base_p = slot_idx * _JWORDS - h0 * _ROW
            sd0, ld0, sd1 = pv[0], pv[1], pv[2]
            in_d = (d < sd0 + ld0, d >= sd1)
            for bd in range(2):
                @pl.when(in_d[bd])
                def _(bd=bd):
                    for bh in range(2):
                        sh = pv[4] if bh == 0 else pv[6]
                        lh = pv[5] if bh == 0 else pv[7]
                        hlo = jnp.maximum(sh, h0)
                        hhi = jnp.minimum(sh + lh, h0 + _JROWS)

                        def h_body(h, c2, bd=bd, bh=bh):
                            hb = base_p + h * _ROW
                            for bw in range(2):
                                sw = pv[8] if bw == 0 else pv[10]
                                lw = pv[9] if bw == 0 else pv[11]
                                ms = wmax8(hb, sw, lw)
                                o = bd * 4 + bh * 2 + bw
                                for cc in range(_CC):
                                    slot = (cc * 8 + o) * _L
                                    acc[pl.ds(slot, _L)] = jnp.maximum(
                                        acc[pl.ds(slot, _L)], ms[cc])
                            return c2

                        lax.fori_loop(hlo, hhi, h_body, 0)
            return None

        def flush_unit(t):
            pn = t * _NW + wid
            for o in range(8 * _CC):
                outbuf[pl.ds(t * 1024 + o * _L, _L)] = acc[pl.ds(o * _L, _L)]
            pltpu.async_copy(
                outbuf.at[pl.ds(t * 1024, 1024)],
                out_hbm.at[pl.ds(pn * 1024, 1024)],
                osem)
            for o in range(8 * _CC):
                acc[pl.ds(o * _L, _L)] = neg

        for o in range(8 * _CC):
            acc[pl.ds(o * _L, _L)] = neg

        pv0, b0 = load_pv(0)
        ti, di, ki, pvi, bi = i32(0), pv0[0], i32(0), pv0, b0
        for s in range(_NRING):
            issue_job_static(bi, di, h0_of(pvi, ki), s)
            ti, di, ki, pvi, bi, _ = adv(ti, di, ki, pvi, bi)

        def count_body(t, tot):
            pv, _ = load_pv(t)
            nd = pv[2] + pv[3] - pv[0]
            return tot + nd * nk_of(pv)

        total_jobs = lax.fori_loop(0, units_per_w, count_body, i32(0))

        def body(cs, st):
            tc, dc, kc, pvc, bc, ti, di, ki, pvi, bi = st
            slot = lax.bitwise_and(cs, i32(_NRING - 1))
            more = ti < units_per_w
            h0i = h0_of(pvi, ki)
            for s in range(_NRING):
                @pl.when(slot == s)
                def _(s=s):
                    wait_job_static(s)
            compute_job(pvc, dc, kc, slot)
            for s in range(_NRING):
                @pl.when(jnp.logical_and(slot == s, more))
                def _(s=s):
                    issue_job_static(bi, di, h0i, s)
            ti2, di2, ki2, pvi2, bi2, _ = adv(ti, di, ki, pvi, bi)
            tc2, dc2, kc2, pvc2, bc2, rolled = adv(tc, dc, kc, pvc, bc)

            @pl.when(rolled)
            def _():
                flush_unit(tc)

            return (tc2, dc2, kc2, pvc2, bc2,
                    ti2, di2, ki2, pvi2, bi2)

        pvc0, bc0 = load_pv(0)
        lax.fori_loop(0, total_jobs, body, (i32(0), pvc0[0], i32(0), pvc0,
                                            bc0, ti, di, ki, pvi, bi))

        def drain(t, carry):
            pltpu.make_async_copy(
                outbuf.at[pl.ds(0, 1024)],
                out_hbm.at[pl.ds(0, 1024)],
                osem).wait()
            return carry

        lax.fori_loop(0, units_per_w, drain, 0)

    return k


def kernel(fm, corners, scale):
    B, C, D, H, W = fm.shape
    N = corners.shape[1]

    c32 = corners.astype(jnp.int32)
    p1 = jnp.clip(c32[:, :, 0, :] // scale, 0, 21)
    p2r = c32[:, :, 1, :] // scale
    p2 = jnp.where(p2r - p1 >= 2, p2r, p1 + 2)
    n = p2 - p1
    s0, s1 = p1, p1 + n // 2
    l0, l1 = (n + 1) // 2, n - n // 2
    pr = jnp.stack(
        [s0[..., 0], l0[..., 0], s1[..., 0], l1[..., 0],
         s0[..., 1], l0[..., 1], s1[..., 1], l1[..., 1],
         s0[..., 2], l0[..., 2], s1[..., 2], l1[..., 2]], axis=-1)
    params = jnp.concatenate(
        [pr, jnp.zeros((B, N, 4), jnp.int32)], axis=-1).reshape(B * N * _L)

    fm2 = _tc_channel_minor(fm)
    out = _build_sc_kernel(B * N)(fm2, params)
    out = _tc_oct_minor(out, B, N).reshape(B, N, C, 2, 2, 2)
    return out


# speedup vs baseline: 116.5448x; 1.0013x over previous
import functools

import jax
import jax.numpy as jnp
from jax import lax
from jax.experimental import pallas as pl
from jax.experimental.pallas import tpu as pltpu
from jax.experimental.pallas import tpu_sc as plsc

_D = _H = _W = 24
_C = 128
_L = 16
_CC = _C // _L
_NW = 32
_ROW = _W * _C
_SLAB = _H * _ROW
_JROWS = 8
_JWORDS = _JROWS * _ROW
_NRING = 4


def _tp_body(x_ref, o_ref):
    o_ref[0] = x_ref[0, :, 0].reshape(_C, _H * _W).T


def _tc_channel_minor(fm):
    B, C, D, H, W = fm.shape
    out = pl.pallas_call(
        _tp_body,
        grid=(B, D),
        in_specs=[pl.BlockSpec((1, C, 1, H, W),
                               lambda b, d: (b, 0, d, 0, 0))],
        out_specs=pl.BlockSpec((1, H * W, C), lambda b, d: (b * D + d, 0, 0)),
        out_shape=jax.ShapeDtypeStruct((B * D, H * W, C), jnp.float32),
    )(fm)
    return out.reshape(B * D * H * W * C)


def _ot_body(x_ref, o_ref):
    o_ref[...] = jnp.swapaxes(x_ref[...], 2, 3)


def _tc_oct_minor(flat, B, N):
    x = flat.reshape(B * N, _CC, 8, _L)
    out = pl.pallas_call(
        _ot_body,
        grid=(1,),
        in_specs=[pl.BlockSpec((B * N, _CC, 8, _L), lambda i: (0, 0, 0, 0))],
        out_specs=pl.BlockSpec((B * N, _CC, _L, 8), lambda i: (0, 0, 0, 0)),
        out_shape=jax.ShapeDtypeStruct((B * N, _CC, _L, 8), jnp.float32),
    )(x)
    return out


def _build_sc_kernel(num_props):
    mesh = plsc.VectorSubcoreMesh(core_axis_name="c", subcore_axis_name="s")
    units_per_w = num_props // _NW

    @functools.partial(
        pl.kernel,
        mesh=mesh,
        out_type=jax.ShapeDtypeStruct((num_props * 8 * _C,), jnp.float32),
        scratch_types=[
            pltpu.VMEM((num_props * _L,), jnp.int32),
            pltpu.VMEM((_NRING * _JWORDS + 512,), jnp.float32),
            pltpu.VMEM((8 * _C,), jnp.float32),
            pltpu.VMEM((units_per_w * 8 * _C,), jnp.float32),
        ] + [pltpu.SemaphoreType.DMA] * (_NRING + 1),
    )
    def k(fm_hbm, par_hbm, out_hbm, par_v, ring, acc, outbuf, *sems_all):
        sems = sems_all[:_NRING]
        osem = sems_all[_NRING]
        wid = lax.axis_index("s") * 2 + lax.axis_index("c")
        pltpu.sync_copy(par_hbm, par_v)
        neg = jnp.full((_L,), -jnp.inf, jnp.float32)
        i32 = jnp.int32

        def load_pv(t):
            ts = jnp.minimum(t, units_per_w - 1)
            pn = ts * _NW + wid
            b = lax.shift_right_logical(pn, 6)
            return par_v[pl.ds(pn * _L, _L)], b

        def nk_of(pv):
            nh = pv[6] + pv[7] - pv[4]
            return lax.shift_right_logical(nh + 7, 3)

        def h0_of(pv, kk):
            return jnp.minimum(pv[4] + kk * _JROWS, _H - _JROWS)

        def issue_job_static(b, d, h0, s):
            src = (b * _D + d) * _SLAB + h0 * _ROW
            pltpu.async_copy(
                fm_hbm.at[pl.ds(src, _JWORDS)],
                ring.at[pl.ds(s * _JWORDS, _JWORDS)],
                sems[s])

        def wait_job_static(s):
            pltpu.make_async_copy(
                fm_hbm.at[pl.ds(0, _JWORDS)],
                ring.at[pl.ds(s * _JWORDS, _JWORDS)],
                sems[s]).wait()

        def adv(t, d, kk, pv, b):
            k2 = kk + 1
            roll_k = k2 >= nk_of(pv)
            k3 = jnp.where(roll_k, 0, k2)
            d2 = jnp.where(roll_k, d + 1, d)
            d_hi = pv[2] + pv[3]
            roll_u = jnp.logical_and(roll_k, d2 >= d_hi)
            t2 = jnp.where(roll_u, t + 1, t)
            pv2, b2 = load_pv(t2)
            pv3 = jnp.where(roll_u, pv2, pv)
            b3 = jnp.where(roll_u, b2, b)
            d3 = jnp.where(roll_u, pv3[0], d2)
            return t2, d3, k3, pv3, b3, roll_u

        def wmax8(hb, sw, lw):
            w_end = sw + lw
            nch = lax.shift_right_logical(lw + 3, 2)

            def chunk(j, ms):
                w0 = sw + j * 4
                base = hb + w0 * _C
                out = []
                for cc in range(_CC):
                    m = ms[cc]
                    for e in range(4):
                        x = ring[pl.ds(base + e * _C + cc * _L, _L)]
                        if e == 0:
                            m = jnp.maximum(m, x)
                        else:
                            m = jnp.maximum(
                                m, jnp.where(w0 + e < w_end, x, neg))
                    out.append(m)
                return tuple(out)

            ms = lax.fori_loop(0, nch, chunk, (neg,) * _CC)
            return ms

        def compute_job(pv, d, kk, slot_idx):
            h0 = h0_of(pv, kk)
            base_p = slot_idx * _JWORDS - h0 * _ROW
            sd0, ld0, sd1 = pv[0], pv[1], pv[2]
            in_d = (d < sd0 + ld0, d >= sd1)
            for bd in range(2):
                @pl.when(in_d[bd])
                def _(bd=bd):
                    for bh in range(2):
                        sh = pv[4] if bh == 0 else pv[6]
                        lh = pv[5] if bh == 0 else pv[7]
                        hlo = jnp.maximum(sh, h0)
                        hhi = jnp.minimum(sh + lh, h0 + _JROWS)

                        def h_body(h, c2, bd=bd, bh=bh):
                            hb = base_p + h * _ROW
                            for bw in range(2):
                                sw = pv[8] if bw == 0 else pv[10]
                                lw = pv[9] if bw == 0 else pv[11]
                                ms = wmax8(hb, sw, lw)
                                o = bd * 4 + bh * 2 + bw
                                for cc in range(_CC):
                                    slot = (cc * 8 + o) * _L
                                    acc[pl.ds(slot, _L)] = jnp.maximum(
                                        acc[pl.ds(slot, _L)], ms[cc])
                            return c2

                        lax.fori_loop(hlo, hhi, h_body, 0)
            return None

        def flush_unit(t):
            pn = t * _NW + wid
            for o in range(8 * _CC):
                outbuf[pl.ds(t * 1024 + o * _L, _L)] = acc[pl.ds(o * _L, _L)]
            pltpu.async_copy(
                outbuf.at[pl.ds(t * 1024, 1024)],
                out_hbm.at[pl.ds(pn * 1024, 1024)],
                osem)
            for o in range(8 * _CC):
                acc[pl.ds(o * _L, _L)] = neg

        for o in range(8 * _CC):
            acc[pl.ds(o * _L, _L)] = neg

        pv0, b0 = load_pv(0)
        ti, di, ki, pvi, bi = i32(0), pv0[0], i32(0), pv0, b0
        for s in range(_NRING):
            issue_job_static(bi, di, h0_of(pvi, ki), s)
            ti, di, ki, pvi, bi, _ = adv(ti, di, ki, pvi, bi)

        def count_body(t, tot):
            pv, _ = load_pv(t)
            nd = pv[2] + pv[3] - pv[0]
            return tot + nd * nk_of(pv)

        total_jobs = lax.fori_loop(0, units_per_w, count_body, i32(0))

        def body(cs, st):
            tc, dc, kc, pvc, bc, ti, di, ki, pvi, bi = st
            slot = lax.bitwise_and(cs, i32(_NRING - 1))
            more = ti < units_per_w
            h0i = h0_of(pvi, ki)
            for s in range(_NRING):
                @pl.when(slot == s)
                def _(s=s):
                    wait_job_static(s)
            compute_job(pvc, dc, kc, slot)
            for s in range(_NRING):
                @pl.when(jnp.logical_and(slot == s, more))
                def _(s=s):
                    issue_job_static(bi, di, h0i, s)
            ti2, di2, ki2, pvi2, bi2, _ = adv(ti, di, ki, pvi, bi)
            tc2, dc2, kc2, pvc2, bc2, rolled = adv(tc, dc, kc, pvc, bc)

            @pl.when(rolled)
            def _():
                flush_unit(tc)

            return (tc2, dc2, kc2, pvc2, bc2,
                    ti2, di2, ki2, pvi2, bi2)

        pvc0, bc0 = load_pv(0)
        lax.fori_loop(0, total_jobs, body, (i32(0), pvc0[0], i32(0), pvc0,
                                            bc0, ti, di, ki, pvi, bi))

        def drain(t, carry):
            pltpu.make_async_copy(
                outbuf.at[pl.ds(0, 1024)],
                out_hbm.at[pl.ds(0, 1024)],
                osem).wait()
            return carry

        lax.fori_loop(0, units_per_w, drain, 0)

    return k


def kernel(fm, corners, scale):
    B, C, D, H, W = fm.shape
    N = corners.shape[1]

    c32 = corners.astype(jnp.int32)
    p1 = jnp.clip(c32[:, :, 0, :] // scale, 0, 21)
    p2r = c32[:, :, 1, :] // scale
    p2 = jnp.where(p2r - p1 >= 2, p2r, p1 + 2)
    n = p2 - p1
    s0, s1 = p1, p1 + n // 2
    l0, l1 = (n + 1) // 2, n - n // 2
    pr = jnp.stack(
        [s0[..., 0], l0[..., 0], s1[..., 0], l1[..., 0],
         s0[..., 1], l0[..., 1], s1[..., 1], l1[..., 1],
         s0[..., 2], l0[..., 2], s1[..., 2], l1[..., 2]], axis=-1)
    params = jnp.concatenate(
        [pr, jnp.zeros((B, N, 4), jnp.int32)], axis=-1).reshape(B * N * _L)

    fm2 = _tc_channel_minor(fm)
    out = _build_sc_kernel(B * N)(fm2, params)
    out = _tc_oct_minor(out, B, N).reshape(B, N, C, 2, 2, 2)
    return out


# speedup vs baseline: 155.2943x; 1.3325x over previous
import functools

import jax
import jax.numpy as jnp
from jax import lax
from jax.experimental import pallas as pl
from jax.experimental.pallas import tpu as pltpu
from jax.experimental.pallas import tpu_sc as plsc

_D = _H = _W = 24
_C = 128
_L = 16
_CC = _C // _L
_NW = 32
_ROW = _W * _C
_SLAB = _H * _ROW
_JROWS = 4
_JWORDS = _JROWS * _ROW
_NRING = 8


_DBLK = 4


def _tp_body(x_ref, o_ref):
    for d in range(_DBLK):
        o_ref[d] = x_ref[0, :, d].reshape(_C, _H * _W).T


def _tc_channel_minor(fm):
    B, C, D, H, W = fm.shape
    out = pl.pallas_call(
        _tp_body,
        grid=(B, D // _DBLK),
        in_specs=[pl.BlockSpec((1, C, _DBLK, H, W),
                               lambda b, j: (b, 0, j, 0, 0))],
        out_specs=pl.BlockSpec((_DBLK, H * W, C),
                               lambda b, j: (b * (D // _DBLK) + j, 0, 0)),
        out_shape=jax.ShapeDtypeStruct((B * D, H * W, C), jnp.float32),
    )(fm)
    return out.reshape(B * D * H * W * C)


def _ot_body(x_ref, o_ref):
    o_ref[...] = jnp.swapaxes(x_ref[...], 2, 3)


def _tc_oct_minor(flat, B, N):
    x = flat.reshape(B * N, _CC, 8, _L)
    out = pl.pallas_call(
        _ot_body,
        grid=(1,),
        in_specs=[pl.BlockSpec((B * N, _CC, 8, _L), lambda i: (0, 0, 0, 0))],
        out_specs=pl.BlockSpec((B * N, _CC, _L, 8), lambda i: (0, 0, 0, 0)),
        out_shape=jax.ShapeDtypeStruct((B * N, _CC, _L, 8), jnp.float32),
    )(x)
    return out


def _build_sc_kernel(num_props):
    mesh = plsc.VectorSubcoreMesh(core_axis_name="c", subcore_axis_name="s")
    units_per_w = num_props // _NW

    @functools.partial(
        pl.kernel,
        mesh=mesh,
        out_type=jax.ShapeDtypeStruct((num_props * 8 * _C,), jnp.float32),
        scratch_types=[
            pltpu.VMEM((num_props * _L,), jnp.int32),
            pltpu.VMEM((_NRING * _JWORDS + 512,), jnp.float32),
            pltpu.VMEM((8 * _C,), jnp.float32),
            pltpu.VMEM((units_per_w * 8 * _C,), jnp.float32),
        ] + [pltpu.SemaphoreType.DMA] * (_NRING + 1),
    )
    def k(fm_hbm, par_hbm, out_hbm, par_v, ring, acc, outbuf, *sems_all):
        sems = sems_all[:_NRING]
        osem = sems_all[_NRING]
        wid = lax.axis_index("s") * 2 + lax.axis_index("c")
        pltpu.sync_copy(par_hbm, par_v)
        neg = jnp.full((_L,), -jnp.inf, jnp.float32)
        i32 = jnp.int32

        def load_pv(t):
            ts = jnp.minimum(t, units_per_w - 1)
            pn = ts * _NW + wid
            b = lax.shift_right_logical(pn, 6)
            return par_v[pl.ds(pn * _L, _L)], b

        def nk_of(pv):
            nh = pv[6] + pv[7] - pv[4]
            return lax.shift_right_logical(nh + 7, 3)

        def h0_of(pv, kk):
            return jnp.minimum(pv[4] + kk * _JROWS, _H - _JROWS)

        def issue_job_static(b, d, h0, s):
            src = (b * _D + d) * _SLAB + h0 * _ROW
            pltpu.async_copy(
                fm_hbm.at[pl.ds(src, _JWORDS)],
                ring.at[pl.ds(s * _JWORDS, _JWORDS)],
                sems[s])

        def wait_job_static(s):
            pltpu.make_async_copy(
                fm_hbm.at[pl.ds(0, _JWORDS)],
                ring.at[pl.ds(s * _JWORDS, _JWORDS)],
                sems[s]).wait()

        def adv(t, d, kk, pv, b):
            k2 = kk + 1
            roll_k = k2 >= nk_of(pv)
            k3 = jnp.where(roll_k, 0, k2)
            d2 = jnp.where(roll_k, d + 1, d)
            d_hi = pv[2] + pv[3]
            roll_u = jnp.logical_and(roll_k, d2 >= d_hi)
            t2 = jnp.where(roll_u, t + 1, t)
            pv2, b2 = load_pv(t2)
            pv3 = jnp.where(roll_u, pv2, pv)
            b3 = jnp.where(roll_u, b2, b)
            d3 = jnp.where(roll_u, pv3[0], d2)
            return t2, d3, k3, pv3, b3, roll_u

        def wmax8(hb, sw, lw):
            w_end = sw + lw
            nch = lax.shift_right_logical(lw + 3, 2)

            def chunk(j, ms):
                w0 = sw + j * 4
                base = hb + w0 * _C
                out = []
                for cc in range(_CC):
                    m = ms[cc]
                    for e in range(4):
                        x = ring[pl.ds(base + e * _C + cc * _L, _L)]
                        if e == 0:
                            m = jnp.maximum(m, x)
                        else:
                            m = jnp.maximum(
                                m, jnp.where(w0 + e < w_end, x, neg))
                    out.append(m)
                return tuple(out)

            ms = lax.fori_loop(0, nch, chunk, (neg,) * _CC)
            return ms

        def compute_job(pv, d, kk, slot_idx):
            h0 = h0_of(pv, kk)
            base_p = slot_idx * _JWORDS - h0 * _ROW
            sd0, ld0, sd1 = pv[0], pv[1], pv[2]
            in_d = (d < sd0 + ld0, d >= sd1)
            for bd in range(2):
                @pl.when(in_d[bd])
                def _(bd=bd):
                    for bh in range(2):
                        sh = pv[4] if bh == 0 else pv[6]
                        lh = pv[5] if bh == 0 else pv[7]
                        hlo = jnp.maximum(sh, h0)
                        hhi = jnp.minimum(sh + lh, h0 + _JROWS)

                        def h_body(h, c2, bd=bd, bh=bh):
                            hb = base_p + h * _ROW
                            for bw in range(2):
                                sw = pv[8] if bw == 0 else pv[10]
                                lw = pv[9] if bw == 0 else pv[11]
                                ms = wmax8(hb, sw, lw)
                                o = bd * 4 + bh * 2 + bw
                                for cc in range(_CC):
                                    slot = (cc * 8 + o) * _L
                                    acc[pl.ds(slot, _L)] = jnp.maximum(
                                        acc[pl.ds(slot, _L)], ms[cc])
                            return c2

                        lax.fori_loop(hlo, hhi, h_body, 0)
            return None

        def flush_unit(t):
            pn = t * _NW + wid
            for o in range(8 * _CC):
                outbuf[pl.ds(t * 1024 + o * _L, _L)] = acc[pl.ds(o * _L, _L)]
            pltpu.async_copy(
                outbuf.at[pl.ds(t * 1024, 1024)],
                out_hbm.at[pl.ds(pn * 1024, 1024)],
                osem)
            for o in range(8 * _CC):
                acc[pl.ds(o * _L, _L)] = neg

        for o in range(8 * _CC):
            acc[pl.ds(o * _L, _L)] = neg

        pv0, b0 = load_pv(0)
        ti, di, ki, pvi, bi = i32(0), pv0[0], i32(0), pv0, b0
        for s in range(_NRING):
            issue_job_static(bi, di, h0_of(pvi, ki), s)
            ti, di, ki, pvi, bi, _ = adv(ti, di, ki, pvi, bi)

        def count_body(t, tot):
            pv, _ = load_pv(t)
            nd = pv[2] + pv[3] - pv[0]
            return tot + nd * nk_of(pv)

        total_jobs = lax.fori_loop(0, units_per_w, count_body, i32(0))

        def body(cs, st):
            tc, dc, kc, pvc, bc, ti, di, ki, pvi, bi = st
            slot = lax.bitwise_and(cs, i32(_NRING - 1))
            more = ti < units_per_w
            h0i = h0_of(pvi, ki)
            for s in range(_NRING):
                @pl.when(slot == s)
                def _(s=s):
                    wait_job_static(s)
            compute_job(pvc, dc, kc, slot)
            for s in range(_NRING):
                @pl.when(jnp.logical_and(slot == s, more))
                def _(s=s):
                    issue_job_static(bi, di, h0i, s)
            ti2, di2, ki2, pvi2, bi2, _ = adv(ti, di, ki, pvi, bi)
            tc2, dc2, kc2, pvc2, bc2, rolled = adv(tc, dc, kc, pvc, bc)

            @pl.when(rolled)
            def _():
                flush_unit(tc)

            return (tc2, dc2, kc2, pvc2, bc2,
                    ti2, di2, ki2, pvi2, bi2)

        pvc0, bc0 = load_pv(0)
        lax.fori_loop(0, total_jobs, body, (i32(0), pvc0[0], i32(0), pvc0,
                                            bc0, ti, di, ki, pvi, bi))

        def drain(t, carry):
            pltpu.make_async_copy(
                outbuf.at[pl.ds(0, 1024)],
                out_hbm.at[pl.ds(0, 1024)],
                osem).wait()
            return carry

        lax.fori_loop(0, units_per_w, drain, 0)

    return k


def kernel(fm, corners, scale):
    B, C, D, H, W = fm.shape
    N = corners.shape[1]

    c32 = corners.astype(jnp.int32)
    p1 = jnp.clip(c32[:, :, 0, :] // scale, 0, 21)
    p2r = c32[:, :, 1, :] // scale
    p2 = jnp.where(p2r - p1 >= 2, p2r, p1 + 2)
    n = p2 - p1
    s0, s1 = p1, p1 + n // 2
    l0, l1 = (n + 1) // 2, n - n // 2
    pr = jnp.stack(
        [s0[..., 0], l0[..., 0], s1[..., 0], l1[..., 0],
         s0[..., 1], l0[..., 1], s1[..., 1], l1[..., 1],
         s0[..., 2], l0[..., 2], s1[..., 2], l1[..., 2]], axis=-1)
    params = jnp.concatenate(
        [pr, jnp.zeros((B, N, 4), jnp.int32)], axis=-1).reshape(B * N * _L)

    fm2 = _tc_channel_minor(fm)
    out = _build_sc_kernel(B * N)(fm2, params)
    out = _tc_oct_minor(out, B, N).reshape(B, N, C, 2, 2, 2)
    return out
